# Initial kernel scaffold; baseline (speedup 1.0000x reference)
import jax
import jax.numpy as jnp
from jax.experimental import pallas as pl


def kernel(x, W1, b1, g1, be1, W2, b2, codebook, W3, b3, g2, be2, W4, b4):
    raise NotImplementedError("write your pallas kernel here")


# speedup vs baseline: 2.1840x; 2.1840x over previous
import functools

import jax
import jax.numpy as jnp
from jax import lax
from jax.experimental import pallas as pl
from jax.experimental.pallas import tpu as pltpu
from jax.experimental.pallas import tpu_sc as plsc

_EPS = 1e-5


def _gelu(h):
    return 0.5 * h * (1.0 + lax.erf(h * 0.7071067811865476))


def _ln(h, g, b):
    mu = jnp.mean(h, axis=-1, keepdims=True)
    var = jnp.mean((h - mu) ** 2, axis=-1, keepdims=True)
    return (h - mu) / jnp.sqrt(var + _EPS) * g + b


def _encode_body(x_ref, w1_ref, b1_ref, g1_ref, be1_ref, w2_ref, b2_ref,
                 cbt_ref, idx_ref, commit_ref, counts_ref):
    i = pl.program_id(0)
    nb = x_ref.shape[0]
    k = cbt_ref.shape[1]
    h = jnp.dot(x_ref[...], w1_ref[...], preferred_element_type=jnp.float32) + b1_ref[...]
    h = _ln(h, g1_ref[...], be1_ref[...])
    h = _gelu(h)
    z = jnp.dot(h, w2_ref[...], preferred_element_type=jnp.float32) + b2_ref[...]
    cbt = cbt_ref[...]
    csq = jnp.sum(cbt * cbt, axis=0, keepdims=True)
    e = csq - 2.0 * jnp.dot(z, cbt, preferred_element_type=jnp.float32)
    idx = jnp.argmin(e, axis=-1, keepdims=True).astype(jnp.int32)
    m = jnp.min(e, axis=-1, keepdims=True)
    zsq = jnp.sum(z * z, axis=-1, keepdims=True)
    idx_ref[...] = idx
    part = jnp.sum(m + zsq)
    onehot = (idx == lax.broadcasted_iota(jnp.int32, (nb, k), 1)).astype(jnp.float32)
    cpart = jnp.sum(onehot, axis=0, keepdims=True)

    @pl.when(i == 0)
    def _():
        commit_ref[0, 0] = 0.0
        counts_ref[...] = jnp.zeros_like(counts_ref)

    commit_ref[0, 0] += part
    counts_ref[...] += cpart


def _decode_body(cb_ref, w3_ref, b3_ref, g2_ref, be2_ref, w4_ref, b4_ref,
                 counts_ref, table_ref, perp_ref, *, n_tokens):
    hq = jnp.dot(cb_ref[...], w3_ref[...], preferred_element_type=jnp.float32) + b3_ref[...]
    hq = _ln(hq, g2_ref[...], be2_ref[...])
    hq = _gelu(hq)
    table_ref[...] = jnp.dot(hq, w4_ref[...], preferred_element_type=jnp.float32) + b4_ref[...]
    p = counts_ref[...] / float(n_tokens)
    ent = -jnp.sum(p * jnp.log(p + 1e-10))
    perp_ref[0, 0] = jnp.exp(ent)


def _make_sc_gather(n, dp, n_workers, chunk):
    b_per_w = n // n_workers
    n_chunks = b_per_w // chunk
    mesh = plsc.VectorSubcoreMesh(core_axis_name="c", subcore_axis_name="s")

    @functools.partial(
        pl.kernel,
        mesh=mesh,
        out_type=jax.ShapeDtypeStruct((n, dp), jnp.float32),
        scratch_types=[
            pltpu.VMEM((b_per_w,), jnp.int32),
            pltpu.VMEM((b_per_w, dp), jnp.float32),
            pltpu.SemaphoreType.DMA,
        ],
        compiler_params=pltpu.CompilerParams(use_tc_tiling_on_sc=False),
    )
    def sc_gather(table_hbm, idx_hbm, out_hbm, idx_v, rows_v, sem):
        wid = lax.axis_index("s") * 2 + lax.axis_index("c")
        base = wid * b_per_w
        pltpu.sync_copy(idx_hbm.at[pl.ds(base, b_per_w)], idx_v)
        handles = []
        for j in range(n_chunks):
            handles.append(pltpu.async_copy(
                table_hbm.at[idx_v.at[pl.ds(j * chunk, chunk)]],
                rows_v.at[pl.ds(j * chunk, chunk)],
                sem,
            ))
        for hnd in handles:
            hnd.wait()
        pltpu.sync_copy(rows_v, out_hbm.at[pl.ds(base, b_per_w)])

    return sc_gather


def kernel(x, W1, b1, g1, be1, W2, b2, codebook, W3, b3, g2, be2, W4, b4):
    B, T, D = x.shape
    K, L = codebook.shape
    H = W1.shape[1]
    N = B * T
    NB = 4096
    grid = N // NB
    DP = 16

    xf = x.reshape(N, D)
    cbt = codebook.T

    rep = lambda shape: pl.BlockSpec(shape, lambda i: (0, 0))
    rep0 = lambda shape: pl.BlockSpec(shape, lambda: (0, 0))
    idx_col, commit_sum, counts = pl.pallas_call(
        _encode_body,
        grid=(grid,),
        in_specs=[
            pl.BlockSpec((NB, D), lambda i: (i, 0)),
            rep((D, H)), rep((1, H)), rep((1, H)), rep((1, H)),
            rep((H, L)), rep((1, L)),
            rep((L, K)),
        ],
        out_specs=[
            pl.BlockSpec((NB, 1), lambda i: (i, 0)),
            pl.BlockSpec((1, 1), lambda i: (0, 0), memory_space=pltpu.SMEM),
            pl.BlockSpec((1, K), lambda i: (0, 0)),
        ],
        out_shape=[
            jax.ShapeDtypeStruct((N, 1), jnp.int32),
            jax.ShapeDtypeStruct((1, 1), jnp.float32),
            jax.ShapeDtypeStruct((1, K), jnp.float32),
        ],
        compiler_params=pltpu.CompilerParams(
            dimension_semantics=("arbitrary",),
        ),
    )(xf, W1, b1.reshape(1, H), g1.reshape(1, H), be1.reshape(1, H),
      W2, b2.reshape(1, L), cbt)

    W4p = jnp.pad(W4, ((0, 0), (0, DP - D)))
    b4p = jnp.pad(b4, (0, DP - D)).reshape(1, DP)
    table, perp = pl.pallas_call(
        functools.partial(_decode_body, n_tokens=N),
        in_specs=[
            rep0((K, L)), rep0((L, H)), rep0((1, H)), rep0((1, H)), rep0((1, H)),
            rep0((H, DP)), rep0((1, DP)), rep0((1, K)),
        ],
        out_specs=[
            pl.BlockSpec((K, DP), lambda: (0, 0)),
            pl.BlockSpec((1, 1), lambda: (0, 0), memory_space=pltpu.SMEM),
        ],
        out_shape=[
            jax.ShapeDtypeStruct((K, DP), jnp.float32),
            jax.ShapeDtypeStruct((1, 1), jnp.float32),
        ],
    )(codebook, W3, b3.reshape(1, H), g2.reshape(1, H), be2.reshape(1, H),
      W4p, b4p, counts)

    idx_flat = idx_col.reshape(N)
    out16 = _make_sc_gather(N, DP, 32, 128)(table, idx_flat)

    reconstructed = out16[:, :D].reshape(B, T, D)
    indices = idx_col.reshape(B, T)
    commit_loss = (0.25 / (N * L)) * commit_sum[0, 0]
    perplexity = perp[0, 0]
    return (reconstructed, indices, commit_loss, perplexity)


# speedup vs baseline: 2.9673x; 1.3587x over previous
import functools

import jax
import jax.numpy as jnp
from jax import lax
from jax.experimental import pallas as pl
from jax.experimental.pallas import tpu as pltpu
from jax.experimental.pallas import tpu_sc as plsc

_EPS = 1e-5


def _gelu(h):
    return 0.5 * h * (1.0 + lax.erf(h * 0.7071067811865476))


def _ln(h, g, b):
    mu = jnp.mean(h, axis=-1, keepdims=True)
    var = jnp.mean((h - mu) ** 2, axis=-1, keepdims=True)
    return (h - mu) / jnp.sqrt(var + _EPS) * g + b


def _encode_body(x_ref, w1_ref, b1_ref, g1_ref, be1_ref, w2_ref, b2_ref,
                 cbt_ref, idx_ref, commit_ref, counts_ref):
    i = pl.program_id(0)
    nb = x_ref.shape[0]
    k = cbt_ref.shape[1]
    l = cbt_ref.shape[0]
    h = jnp.dot(x_ref[...], w1_ref[...], preferred_element_type=jnp.float32) + b1_ref[...]
    h = _ln(h, g1_ref[...], be1_ref[...])
    h = _gelu(h)
    z = jnp.dot(h, w2_ref[...], preferred_element_type=jnp.float32) + b2_ref[...]
    cbt = cbt_ref[...]
    csq = jnp.sum(cbt * cbt, axis=0, keepdims=True)
    e = csq - 2.0 * jnp.dot(z, cbt, preferred_element_type=jnp.float32)
    m = jnp.min(e, axis=-1, keepdims=True)
    eq = e == m
    iota_k = lax.broadcasted_iota(jnp.int32, (nb, k), 1)
    idx = jnp.min(jnp.where(eq, iota_k, k), axis=-1, keepdims=True)
    idx_ref[...] = idx
    f = eq.astype(jnp.float32)
    ones_row = jnp.ones((1, nb), jnp.float32)
    cpart = jnp.dot(ones_row, f, preferred_element_type=jnp.float32)
    zsq = jnp.dot(z * z, jnp.ones((l, 1), jnp.float32),
                  preferred_element_type=jnp.float32)
    part = jnp.dot(ones_row, m + zsq, preferred_element_type=jnp.float32)

    @pl.when(i == 0)
    def _():
        commit_ref[0, 0] = 0.0
        counts_ref[...] = jnp.zeros_like(counts_ref)

    commit_ref[0, 0] += part[0, 0]
    counts_ref[...] += cpart


def _decode_body(cb_ref, w3_ref, b3_ref, g2_ref, be2_ref, w4_ref, b4_ref,
                 counts_ref, table_ref, perp_ref, *, n_tokens):
    hq = jnp.dot(cb_ref[...], w3_ref[...], preferred_element_type=jnp.float32) + b3_ref[...]
    hq = _ln(hq, g2_ref[...], be2_ref[...])
    hq = _gelu(hq)
    table_ref[...] = jnp.dot(hq, w4_ref[...], preferred_element_type=jnp.float32) + b4_ref[...]
    p = counts_ref[...] / float(n_tokens)
    ent = -jnp.sum(p * jnp.log(p + 1e-10))
    perp_ref[0, 0] = jnp.exp(ent)


def _make_sc_gather(n, k, dp, n_workers):
    b_per_w = n // n_workers
    groups = b_per_w // 16
    mesh = plsc.VectorSubcoreMesh(core_axis_name="c", subcore_axis_name="s")

    @functools.partial(
        pl.kernel,
        mesh=mesh,
        out_type=jax.ShapeDtypeStruct((n * dp,), jnp.float32),
        scratch_types=[
            pltpu.VMEM((b_per_w,), jnp.int32),
            pltpu.VMEM((k * dp,), jnp.float32),
            pltpu.VMEM((b_per_w * dp,), jnp.float32),
        ],
        compiler_params=pltpu.CompilerParams(
            use_tc_tiling_on_sc=False, needs_layout_passes=False),
    )
    def sc_gather(table_hbm, idx_hbm, out_hbm, idx_v, table_v, rows_v):
        wid = lax.axis_index("s") * 2 + lax.axis_index("c")
        base = wid * b_per_w
        pltpu.sync_copy(table_hbm, table_v)
        pltpu.sync_copy(idx_hbm.at[pl.ds(base, b_per_w)], idx_v)
        lane16 = lax.iota(jnp.int32, 16) * dp

        def body(g, carry):
            idx16 = idx_v[pl.ds(g * 16, 16)]
            rowbase = idx16 * dp
            outbase = lane16 + g * (16 * dp)
            for c in range(dp):
                vals = plsc.load_gather(table_v, [rowbase + c])
                plsc.store_scatter(rows_v, [outbase + c], vals)
            return carry

        lax.fori_loop(0, groups, body, 0)
        pltpu.sync_copy(rows_v, out_hbm.at[pl.ds(base * dp, b_per_w * dp)])

    return sc_gather


def kernel(x, W1, b1, g1, be1, W2, b2, codebook, W3, b3, g2, be2, W4, b4):
    B, T, D = x.shape
    K, L = codebook.shape
    H = W1.shape[1]
    N = B * T
    NB = 4096
    grid = N // NB
    DP = 16

    xf = x.reshape(N, D)
    cbt = codebook.T

    rep = lambda shape: pl.BlockSpec(shape, lambda i: (0, 0))
    rep0 = lambda shape: pl.BlockSpec(shape, lambda: (0, 0))
    idx_col, commit_sum, counts = pl.pallas_call(
        _encode_body,
        grid=(grid,),
        in_specs=[
            pl.BlockSpec((NB, D), lambda i: (i, 0)),
            rep((D, H)), rep((1, H)), rep((1, H)), rep((1, H)),
            rep((H, L)), rep((1, L)),
            rep((L, K)),
        ],
        out_specs=[
            pl.BlockSpec((NB, 1), lambda i: (i, 0)),
            pl.BlockSpec((1, 1), lambda i: (0, 0), memory_space=pltpu.SMEM),
            pl.BlockSpec((1, K), lambda i: (0, 0)),
        ],
        out_shape=[
            jax.ShapeDtypeStruct((N, 1), jnp.int32),
            jax.ShapeDtypeStruct((1, 1), jnp.float32),
            jax.ShapeDtypeStruct((1, K), jnp.float32),
        ],
        compiler_params=pltpu.CompilerParams(
            dimension_semantics=("arbitrary",),
        ),
    )(xf, W1, b1.reshape(1, H), g1.reshape(1, H), be1.reshape(1, H),
      W2, b2.reshape(1, L), cbt)

    W4p = jnp.pad(W4, ((0, 0), (0, DP - D)))
    b4p = jnp.pad(b4, (0, DP - D)).reshape(1, DP)
    table, perp = pl.pallas_call(
        functools.partial(_decode_body, n_tokens=N),
        in_specs=[
            rep0((K, L)), rep0((L, H)), rep0((1, H)), rep0((1, H)), rep0((1, H)),
            rep0((H, DP)), rep0((1, DP)), rep0((1, K)),
        ],
        out_specs=[
            pl.BlockSpec((K, DP), lambda: (0, 0)),
            pl.BlockSpec((1, 1), lambda: (0, 0), memory_space=pltpu.SMEM),
        ],
        out_shape=[
            jax.ShapeDtypeStruct((K, DP), jnp.float32),
            jax.ShapeDtypeStruct((1, 1), jnp.float32),
        ],
    )(codebook, W3, b3.reshape(1, H), g2.reshape(1, H), be2.reshape(1, H),
      W4p, b4p, counts)

    idx_flat = idx_col.reshape(N)
    out_flat = _make_sc_gather(N, K, DP, 32)(table.reshape(K * DP), idx_flat)

    reconstructed = out_flat.reshape(N, DP)[:, :D].reshape(B, T, D)
    indices = idx_col.reshape(B, T)
    commit_loss = (0.25 / (N * L)) * commit_sum[0, 0]
    perplexity = perp[0, 0]
    return (reconstructed, indices, commit_loss, perplexity)
